# baseline (speedup 1.0000x reference)
import functools

import jax
import jax.numpy as jnp
from jax import lax
from jax.experimental import pallas as pl
from jax.experimental.pallas import tpu as pltpu
from jax.experimental.pallas import tpu_sc as plsc

VOCAB = 1000000
DIM = 64
BATCH = 4096
HIST = 200

NC, NS = 2, 16
NW = NC * NS
BC = BATCH // NW
N_UNITS = HIST
NB = 8
AHEAD = 4
NT = 4
N_OUTER = N_UNITS // NB


def _gather_kernel(table_hbm, ids_hbm, out_hbm, idx_v, bufs, gsems, ssems, tbufs):
    wid = lax.axis_index("s") * NC + lax.axis_index("c")
    b0 = wid * BC

    pltpu.sync_copy(ids_hbm.at[:, pl.ds(b0, BC)], idx_v)

    def fire_gather(h, g):
        pltpu.async_copy(table_hbm.at[idx_v.at[h]], bufs[g], gsems[g])

    def wait_gather(g):
        pltpu.make_async_copy(
            table_hbm.at[pl.ds(0, BC)], bufs[g], gsems[g]
        ).wait()

    def transpose(g, t):
        lanes = lax.iota(jnp.int32, 16)

        def srow(sd, carry):
            rows = lanes + sd * 16
            for f in range(DIM):
                col = jnp.full((16,), f, jnp.int32)
                v = plsc.load_gather(bufs[g], [rows, col])
                tbufs[t][f, pl.ds(sd * 16, 16)] = v
            return carry

        lax.fori_loop(0, BC // 16, srow, 0)

    def start_store(h, t):
        pltpu.async_copy(
            tbufs[t], out_hbm.at[h, :, pl.ds(b0, BC)], ssems[t]
        )

    def wait_store(t):
        pltpu.make_async_copy(
            tbufs[t], out_hbm.at[0, :, pl.ds(b0, BC)], ssems[t]
        ).wait()

    for g in range(AHEAD):
        fire_gather(g, g)

    def body(c, carry):
        for u in range(NB):
            p = c * NB + u
            s = (u + AHEAD) % NB
            if u < AHEAD:
                fire_gather(p + AHEAD, s)
            else:
                @pl.when(c < N_OUTER - 1)
                def _():
                    fire_gather(p + AHEAD, s)
            wait_gather(u)
            t = u % NT
            if u < NT:
                @pl.when(c > 0)
                def _():
                    wait_store(t)
            else:
                wait_store(t)
            transpose(u, t)
            start_store(p, t)
        return carry

    lax.fori_loop(0, N_OUTER, body, 0)

    for t in range(NT):
        wait_store(t)


@jax.jit
def _embedding_lookup(ids_t, table):
    mesh = plsc.VectorSubcoreMesh(
        core_axis_name="c", subcore_axis_name="s",
        num_cores=NC, num_subcores=NS,
    )
    run = pl.kernel(
        _gather_kernel,
        out_type=jax.ShapeDtypeStruct((HIST, DIM, BATCH), jnp.float32),
        mesh=mesh,
        scratch_types=[
            pltpu.VMEM((HIST, BC), jnp.int32),
            [pltpu.VMEM((BC, DIM), jnp.float32) for _ in range(NB)],
            [pltpu.SemaphoreType.DMA for _ in range(NB)],
            [pltpu.SemaphoreType.DMA for _ in range(NT)],
            [pltpu.VMEM((DIM, BC), jnp.float32) for _ in range(NT)],
        ],
        compiler_params=pltpu.CompilerParams(
            use_tc_tiling_on_sc=False, needs_layout_passes=False,
        ),
    )
    return run(table, ids_t)


def kernel(input_ids, table):
    ids_t = input_ids.T
    out_t = _embedding_lookup(ids_t, table)
    return jnp.transpose(out_t, (2, 0, 1))


# speedup vs baseline: 1.0016x; 1.0016x over previous
import functools

import jax
import jax.numpy as jnp
from jax import lax
from jax.experimental import pallas as pl
from jax.experimental.pallas import tpu as pltpu
from jax.experimental.pallas import tpu_sc as plsc

VOCAB = 1000000
DIM = 64
BATCH = 4096
HIST = 200

NC, NS = 2, 16
NW = NC * NS
BC = BATCH // NW
N_UNITS = HIST
NB = 8
AHEAD = 4
NT = 4
N_OUTER = N_UNITS // NB


def _gather_kernel(table_hbm, ids_hbm, out_hbm, idx_v, bufs, gsems, ssems, tbufs):
    wid = lax.axis_index("s") * NC + lax.axis_index("c")
    b0 = wid * BC

    pltpu.sync_copy(ids_hbm.at[:, pl.ds(b0, BC)], idx_v)

    def fire_gather(h, g):
        pltpu.async_copy(table_hbm.at[idx_v.at[h]], bufs[g], gsems[g])

    def wait_gather(g):
        pltpu.make_async_copy(
            table_hbm.at[pl.ds(0, BC)], bufs[g], gsems[g]
        ).wait()

    lanes = lax.iota(jnp.int32, 16)
    ones = jnp.ones((16,), jnp.int32)
    zeros = jnp.zeros((16,), jnp.int32)

    def transpose(g, t):
        def srow(sd, carry):
            rows = lanes + sd * 16
            col = zeros
            for f in range(DIM):
                v = plsc.load_gather(bufs[g], [rows, col])
                tbufs[t][f, pl.ds(sd * 16, 16)] = v
                col = col + ones
            return carry

        lax.fori_loop(0, BC // 16, srow, 0)

    def start_store(h, t):
        pltpu.async_copy(
            tbufs[t], out_hbm.at[h, :, pl.ds(b0, BC)], ssems[t]
        )

    def wait_store(t):
        pltpu.make_async_copy(
            tbufs[t], out_hbm.at[0, :, pl.ds(b0, BC)], ssems[t]
        ).wait()

    for g in range(AHEAD):
        fire_gather(g, g)

    def body(c, carry):
        for u in range(NB):
            p = c * NB + u
            s = (u + AHEAD) % NB
            if u < AHEAD:
                fire_gather(p + AHEAD, s)
            else:
                @pl.when(c < N_OUTER - 1)
                def _():
                    fire_gather(p + AHEAD, s)
            wait_gather(u)
            t = u % NT
            if u < NT:
                @pl.when(c > 0)
                def _():
                    wait_store(t)
            else:
                wait_store(t)
            transpose(u, t)
            start_store(p, t)
        return carry

    lax.fori_loop(0, N_OUTER, body, 0)

    for t in range(NT):
        wait_store(t)


@jax.jit
def _embedding_lookup(ids_t, table):
    mesh = plsc.VectorSubcoreMesh(
        core_axis_name="c", subcore_axis_name="s",
        num_cores=NC, num_subcores=NS,
    )
    run = pl.kernel(
        _gather_kernel,
        out_type=jax.ShapeDtypeStruct((HIST, DIM, BATCH), jnp.float32),
        mesh=mesh,
        scratch_types=[
            pltpu.VMEM((HIST, BC), jnp.int32),
            [pltpu.VMEM((BC, DIM), jnp.float32) for _ in range(NB)],
            [pltpu.SemaphoreType.DMA for _ in range(NB)],
            [pltpu.SemaphoreType.DMA for _ in range(NT)],
            [pltpu.VMEM((DIM, BC), jnp.float32) for _ in range(NT)],
        ],
        compiler_params=pltpu.CompilerParams(
            use_tc_tiling_on_sc=False, needs_layout_passes=False,
        ),
    )
    return run(table, ids_t)


def kernel(input_ids, table):
    ids_t = input_ids.T
    out_t = _embedding_lookup(ids_t, table)
    return jnp.transpose(out_t, (2, 0, 1))


# speedup vs baseline: 1.6291x; 1.6266x over previous
import functools

import jax
import jax.numpy as jnp
from jax import lax
from jax.experimental import pallas as pl
from jax.experimental.pallas import tpu as pltpu
from jax.experimental.pallas import tpu_sc as plsc

VOCAB = 1000000
DIM = 64
BATCH = 4096
HIST = 200

NC, NS = 2, 16
NW = NC * NS
BC = BATCH // NW
N_UNITS = HIST
NB = 8
AHEAD = 4
NT = 4
N_OUTER = N_UNITS // NB


def _gather_kernel(table_hbm, ids_hbm, out_hbm, idx_v, bufs, gsems, ssems, tbufs):
    wid = lax.axis_index("s") * NC + lax.axis_index("c")
    b0 = wid * BC

    pltpu.sync_copy(ids_hbm.at[:, pl.ds(b0, BC)], idx_v)

    def fire_gather(h, g):
        pltpu.async_copy(table_hbm.at[idx_v.at[h]], bufs[g], gsems[g])

    def wait_gather(g):
        pltpu.make_async_copy(
            table_hbm.at[pl.ds(0, BC)], bufs[g], gsems[g]
        ).wait()

    lanes = lax.iota(jnp.int32, 16)

    def transpose(g, t):
        def sblk(i, carry):
            bb = i // 16
            k = i - bb * 16
            rowsb = lanes + bb * 16
            base = (lanes + k) & 15
            for ff in range(DIM // 16):
                colr = base + ff * 16
                v = plsc.load_gather(bufs[g], [rowsb, colr])
                plsc.store_scatter(tbufs[t], [colr, rowsb], v)
            return carry

        lax.fori_loop(0, (BC // 16) * 16, sblk, 0)

    def start_store(h, t):
        pltpu.async_copy(
            tbufs[t], out_hbm.at[h, :, pl.ds(b0, BC)], ssems[t]
        )

    def wait_store(t):
        pltpu.make_async_copy(
            tbufs[t], out_hbm.at[0, :, pl.ds(b0, BC)], ssems[t]
        ).wait()

    for g in range(AHEAD):
        fire_gather(g, g)

    def body(c, carry):
        for u in range(NB):
            p = c * NB + u
            s = (u + AHEAD) % NB
            if u < AHEAD:
                fire_gather(p + AHEAD, s)
            else:
                @pl.when(c < N_OUTER - 1)
                def _():
                    fire_gather(p + AHEAD, s)
            wait_gather(u)
            t = u % NT
            if u < NT:
                @pl.when(c > 0)
                def _():
                    wait_store(t)
            else:
                wait_store(t)
            transpose(u, t)
            start_store(p, t)
        return carry

    lax.fori_loop(0, N_OUTER, body, 0)

    for t in range(NT):
        wait_store(t)


@jax.jit
def _embedding_lookup(ids_t, table):
    mesh = plsc.VectorSubcoreMesh(
        core_axis_name="c", subcore_axis_name="s",
        num_cores=NC, num_subcores=NS,
    )
    run = pl.kernel(
        _gather_kernel,
        out_type=jax.ShapeDtypeStruct((HIST, DIM, BATCH), jnp.float32),
        mesh=mesh,
        scratch_types=[
            pltpu.VMEM((HIST, BC), jnp.int32),
            [pltpu.VMEM((BC, DIM), jnp.float32) for _ in range(NB)],
            [pltpu.SemaphoreType.DMA for _ in range(NB)],
            [pltpu.SemaphoreType.DMA for _ in range(NT)],
            [pltpu.VMEM((DIM, BC), jnp.float32) for _ in range(NT)],
        ],
        compiler_params=pltpu.CompilerParams(
            use_tc_tiling_on_sc=False, needs_layout_passes=False,
        ),
    )
    return run(table, ids_t)


def kernel(input_ids, table):
    ids_t = input_ids.T
    out_t = _embedding_lookup(ids_t, table)
    return jnp.transpose(out_t, (2, 0, 1))


# speedup vs baseline: 1.6565x; 1.0168x over previous
import functools

import jax
import jax.numpy as jnp
from jax import lax
from jax.experimental import pallas as pl
from jax.experimental.pallas import tpu as pltpu
from jax.experimental.pallas import tpu_sc as plsc

VOCAB = 1000000
DIM = 64
BATCH = 4096
HIST = 200

NC, NS = 2, 16
NW = NC * NS
BC = BATCH // NW
N_UNITS = HIST
NB = 8
AHEAD = 4
NT = 4
N_OUTER = N_UNITS // NB


def _gather_kernel(table_hbm, ids_hbm, out_hbm, idx_v, bufs, gsems, ssems, tbufs):
    wid = lax.axis_index("s") * NC + lax.axis_index("c")
    b0 = wid * BC

    pltpu.sync_copy(ids_hbm.at[:, pl.ds(b0, BC)], idx_v)

    def fire_gather(h, g):
        pltpu.async_copy(table_hbm.at[idx_v.at[h]], bufs[g], gsems[g])

    def wait_gather(g):
        pltpu.make_async_copy(
            table_hbm.at[pl.ds(0, BC)], bufs[g], gsems[g]
        ).wait()

    lanes = lax.iota(jnp.int32, 16)

    def transpose(g, t):
        def skrot(k, carry):
            base = (lanes + k) & 15
            for bb in range(BC // 16):
                rowsb = lanes + bb * 16
                for ff in range(DIM // 16):
                    colr = base + ff * 16
                    v = plsc.load_gather(bufs[g], [rowsb, colr])
                    plsc.store_scatter(tbufs[t], [colr, rowsb], v)
            return carry

        lax.fori_loop(0, 16, skrot, 0)

    def start_store(h, t):
        pltpu.async_copy(
            tbufs[t], out_hbm.at[h, :, pl.ds(b0, BC)], ssems[t]
        )

    def wait_store(t):
        pltpu.make_async_copy(
            tbufs[t], out_hbm.at[0, :, pl.ds(b0, BC)], ssems[t]
        ).wait()

    for g in range(AHEAD):
        fire_gather(g, g)

    def body(c, carry):
        for u in range(NB):
            p = c * NB + u
            s = (u + AHEAD) % NB
            if u < AHEAD:
                fire_gather(p + AHEAD, s)
            else:
                @pl.when(c < N_OUTER - 1)
                def _():
                    fire_gather(p + AHEAD, s)
            wait_gather(u)
            t = u % NT
            if u < NT:
                @pl.when(c > 0)
                def _():
                    wait_store(t)
            else:
                wait_store(t)
            transpose(u, t)
            start_store(p, t)
        return carry

    lax.fori_loop(0, N_OUTER, body, 0)

    for t in range(NT):
        wait_store(t)


@jax.jit
def _embedding_lookup(ids_t, table):
    mesh = plsc.VectorSubcoreMesh(
        core_axis_name="c", subcore_axis_name="s",
        num_cores=NC, num_subcores=NS,
    )
    run = pl.kernel(
        _gather_kernel,
        out_type=jax.ShapeDtypeStruct((HIST, DIM, BATCH), jnp.float32),
        mesh=mesh,
        scratch_types=[
            pltpu.VMEM((HIST, BC), jnp.int32),
            [pltpu.VMEM((BC, DIM), jnp.float32) for _ in range(NB)],
            [pltpu.SemaphoreType.DMA for _ in range(NB)],
            [pltpu.SemaphoreType.DMA for _ in range(NT)],
            [pltpu.VMEM((DIM, BC), jnp.float32) for _ in range(NT)],
        ],
        compiler_params=pltpu.CompilerParams(
            use_tc_tiling_on_sc=False, needs_layout_passes=False,
        ),
    )
    return run(table, ids_t)


def kernel(input_ids, table):
    ids_t = input_ids.T
    out_t = _embedding_lookup(ids_t, table)
    return jnp.transpose(out_t, (2, 0, 1))


# speedup vs baseline: 1.7602x; 1.0626x over previous
import functools

import jax
import jax.numpy as jnp
from jax import lax
from jax.experimental import pallas as pl
from jax.experimental.pallas import tpu as pltpu
from jax.experimental.pallas import tpu_sc as plsc

VOCAB = 1000000
DIM = 64
BATCH = 4096
HIST = 200

NC, NS = 2, 16
NW = NC * NS
BC = BATCH // NW
N_UNITS = HIST
NB = 4
AHEAD = 2
NT = 2
N_OUTER = N_UNITS // NB


def _gather_kernel(table_hbm, ids_hbm, out_hbm, idx_v, ihalf, bufs, tbufs,
                   gsems, ssems):
    wid = lax.axis_index("s") * NC + lax.axis_index("c")

    pltpu.sync_copy(ids_hbm.at[:, wid, :], idx_v)

    lanes = lax.iota(jnp.int32, 16)

    def fire_gather(h, g):
        for s in range(BC // 16):
            ihalf[g, pl.ds(s * 16, 16)] = (
                idx_v[h, pl.ds(s * 16, 16)] >> 1
            )
        pltpu.async_copy(table_hbm.at[ihalf.at[g]], bufs[g], gsems[g])

    def wait_gather(g):
        pltpu.make_async_copy(
            table_hbm.at[pl.ds(0, BC)], bufs[g], gsems[g]
        ).wait()

    def transpose(h, g, t):
        def skrot(k, carry):
            base = (lanes + k) & 15
            fis = base & 7
            fts = base >> 3
            for bb in range(BC // 16):
                rowsb = lanes + bb * 16
                half = (idx_v[h, pl.ds(bb * 16, 16)] & 1) * 64
                for ff in range(DIM // 16):
                    colr = base + ff * 16
                    v = plsc.load_gather(bufs[g], [rowsb, colr + half])
                    plsc.store_scatter(
                        tbufs[t], [fts + ff * 2, fis, rowsb], v
                    )
            return carry

        lax.fori_loop(0, 16, skrot, 0)

    def start_store(h, t):
        pltpu.async_copy(
            tbufs[t], out_hbm.at[h, :, wid, :, :], ssems[t]
        )

    def wait_store(t):
        pltpu.make_async_copy(
            tbufs[t], out_hbm.at[0, :, wid, :, :], ssems[t]
        ).wait()

    for g in range(AHEAD):
        fire_gather(g, g)

    def body(c, carry):
        for u in range(NB):
            p = c * NB + u
            s = (u + AHEAD) % NB
            if u < NB - AHEAD:
                fire_gather(p + AHEAD, s)
            else:
                @pl.when(c < N_OUTER - 1)
                def _():
                    fire_gather(p + AHEAD, s)
            wait_gather(u)
            t = u % NT
            if u < NT:
                @pl.when(c > 0)
                def _():
                    wait_store(t)
            else:
                wait_store(t)
            transpose(p, u, t)
            start_store(p, t)
        return carry

    lax.fori_loop(0, N_OUTER, body, 0)

    for t in range(NT):
        wait_store(t)


@jax.jit
def _embedding_lookup(ids3, table2):
    mesh = plsc.VectorSubcoreMesh(
        core_axis_name="c", subcore_axis_name="s",
        num_cores=NC, num_subcores=NS,
    )
    run = pl.kernel(
        _gather_kernel,
        out_type=jax.ShapeDtypeStruct((HIST, DIM // 8, NW, 8, BC), jnp.float32),
        mesh=mesh,
        scratch_types=[
            pltpu.VMEM((HIST, BC), jnp.int32),
            pltpu.VMEM((NB, BC), jnp.int32),
            [pltpu.VMEM((BC, 2 * DIM), jnp.float32) for _ in range(NB)],
            [pltpu.VMEM((DIM // 8, 8, BC), jnp.float32) for _ in range(NT)],
            [pltpu.SemaphoreType.DMA for _ in range(NB)],
            [pltpu.SemaphoreType.DMA for _ in range(NT)],
        ],
        compiler_params=pltpu.CompilerParams(
            use_tc_tiling_on_sc=False, needs_layout_passes=False,
        ),
    )
    return run(table2, ids3)


def kernel(input_ids, table):
    ids3 = input_ids.T.reshape(HIST, NW, BC)
    table2 = table.reshape(VOCAB // 2, 2 * DIM)
    out5 = _embedding_lookup(ids3, table2)
    out = out5.transpose(2, 4, 0, 1, 3).reshape(BATCH, HIST, DIM)
    return out


# speedup vs baseline: 2.1559x; 1.2248x over previous
import functools

import jax
import jax.numpy as jnp
from jax import lax
from jax.experimental import pallas as pl
from jax.experimental.pallas import tpu as pltpu
from jax.experimental.pallas import tpu_sc as plsc

VOCAB = 1000000
DIM = 64
BATCH = 4096
HIST = 200

NC, NS = 2, 16
NW = NC * NS
BC = BATCH // NW
N_UNITS = HIST
NB = 4
AHEAD = 2
NT = 2
N_OUTER = N_UNITS // NB


def _gather_kernel(table_hbm, ids_hbm, out_hbm, idx_v, bufs, tbufs,
                   gsems, ssems):
    wid = lax.axis_index("s") * NC + lax.axis_index("c")

    pltpu.sync_copy(ids_hbm.at[:, wid, :], idx_v)

    lanes = lax.iota(jnp.int32, 16)

    def fire_gather(h, g):
        pltpu.async_copy(table_hbm.at[idx_v.at[h]], bufs[g], gsems[g])

    def wait_gather(g):
        pltpu.make_async_copy(
            table_hbm.at[pl.ds(0, BC)], bufs[g], gsems[g]
        ).wait()

    def transpose(h, g, t):
        def skrot(k, carry):
            base = (lanes + k) & 15
            fis = base & 7
            fts = base >> 3
            for bb in range(BC // 16):
                rowsb = lanes + bb * 16
                for ff in range(DIM // 16):
                    colr = base + ff * 16
                    v = plsc.load_gather(bufs[g], [rowsb, colr])
                    plsc.store_scatter(
                        tbufs[t], [fts + ff * 2, fis, rowsb], v
                    )
            return carry

        lax.fori_loop(0, 16, skrot, 0)

    def start_store(h, t):
        pltpu.async_copy(
            tbufs[t], out_hbm.at[h, :, wid, :, :], ssems[t]
        )

    def wait_store(t):
        pltpu.make_async_copy(
            tbufs[t], out_hbm.at[0, :, wid, :, :], ssems[t]
        ).wait()

    for g in range(AHEAD):
        fire_gather(g, g)

    def body(c, carry):
        for u in range(NB):
            p = c * NB + u
            s = (u + AHEAD) % NB
            if u < NB - AHEAD:
                fire_gather(p + AHEAD, s)
            else:
                @pl.when(c < N_OUTER - 1)
                def _():
                    fire_gather(p + AHEAD, s)
            wait_gather(u)
            t = u % NT
            if u < NT:
                @pl.when(c > 0)
                def _():
                    wait_store(t)
            else:
                wait_store(t)
            transpose(p, u, t)
            start_store(p, t)
        return carry

    lax.fori_loop(0, N_OUTER, body, 0)

    for t in range(NT):
        wait_store(t)


@jax.jit
def _embedding_lookup(ids3, table2):
    mesh = plsc.VectorSubcoreMesh(
        core_axis_name="c", subcore_axis_name="s",
        num_cores=NC, num_subcores=NS,
    )
    run = pl.kernel(
        _gather_kernel,
        out_type=jax.ShapeDtypeStruct((HIST, DIM // 8, NW, 8, BC), jnp.float32),
        mesh=mesh,
        scratch_types=[
            pltpu.VMEM((HIST, BC), jnp.int32),
            [pltpu.VMEM((BC, 2 * DIM), jnp.float32) for _ in range(NB)],
            [pltpu.VMEM((DIM // 8, 8, BC), jnp.float32) for _ in range(NT)],
            [pltpu.SemaphoreType.DMA for _ in range(NB)],
            [pltpu.SemaphoreType.DMA for _ in range(NT)],
        ],
        compiler_params=pltpu.CompilerParams(
            use_tc_tiling_on_sc=False, needs_layout_passes=False,
        ),
    )
    return run(table2, ids3)


def kernel(input_ids, table):
    ids3 = input_ids.T.reshape(HIST, NW, BC)
    table2 = jnp.pad(table, ((0, 0), (0, 2 * DIM - DIM)))
    out5 = _embedding_lookup(ids3, table2)
    out = out5.transpose(2, 4, 0, 1, 3).reshape(BATCH, HIST, DIM)
    return out
